# baseline (speedup 1.0000x reference)
import functools

import jax
import jax.numpy as jnp
from jax import lax
from jax.experimental import pallas as pl
from jax.experimental.pallas import tpu as pltpu
from jax.experimental.pallas import tpu_sc as plsc

_NC = 2
_NS = 16
_NW = _NC * _NS

_CHUNK = 128


@functools.cache
def _make_kernel(V, D, B):
  assert B % _NW == 0
  b_per_w = B // _NW
  assert b_per_w % _CHUNK == 0
  n_chunks = b_per_w // _CHUNK

  mesh = plsc.VectorSubcoreMesh(core_axis_name="c", subcore_axis_name="s")

  @functools.partial(
      pl.kernel,
      mesh=mesh,
      compiler_params=pltpu.CompilerParams(use_tc_tiling_on_sc=False),
      out_type=jax.ShapeDtypeStruct((B, D), jnp.float32),
      scratch_types=[
          pltpu.VMEM((n_chunks, _CHUNK), jnp.int32),
          pltpu.VMEM((b_per_w, D), jnp.float32),
          pltpu.SemaphoreType.DMA,
      ],
  )
  def emb(table_hbm, idx_hbm, out_hbm, idx_v, rows_v, sem):
    wid = lax.axis_index("s") * _NC + lax.axis_index("c")
    base = wid * b_per_w
    pltpu.sync_copy(idx_hbm.at[wid], idx_v)
    copies = []
    for j in range(n_chunks):
      copies.append(
          pltpu.async_copy(
              table_hbm.at[idx_v.at[j]],
              rows_v.at[pl.ds(j * _CHUNK, _CHUNK)],
              sem,
          )
      )
    for c in copies:
      c.wait()
    pltpu.sync_copy(rows_v, out_hbm.at[pl.ds(base, b_per_w)])

  return emb


def kernel(input_ids, weight):
  V, D = weight.shape
  (B,) = input_ids.shape
  emb = _make_kernel(V, D, B)
  idx = input_ids.astype(jnp.int32).reshape(_NW, B // _NW // _CHUNK, _CHUNK)
  return emb(weight, idx)


# speedup vs baseline: 2.3124x; 2.3124x over previous
import functools

import jax
import jax.numpy as jnp
from jax import lax
from jax.experimental import pallas as pl
from jax.experimental.pallas import tpu as pltpu
from jax.experimental.pallas import tpu_sc as plsc

_NC = 2
_NS = 16
_NW = _NC * _NS
_L = 16

_CW = 5


@functools.cache
def _make_kernel(V, D, B):
  assert D == 64 and B % _L == 0
  ncols = V // 128
  tail = V - ncols * 128
  sw = -(-ncols // _NW)
  nch = -(-sw // _CW)
  cmax = ncols - _CW
  c0max = ncols - sw
  vmax = ncols * 128

  mesh = plsc.VectorSubcoreMesh(core_axis_name="c", subcore_axis_name="s")

  nb = B

  @functools.partial(
      pl.kernel,
      mesh=mesh,
      compiler_params=pltpu.CompilerParams(needs_layout_passes=False),
      out_type=jax.ShapeDtypeStruct((B + _L, 128), jnp.float32),
      scratch_types=[
          pltpu.VMEM((4 * nb,), jnp.int32),
          pltpu.VMEM((D, _CW * 128), jnp.float32),
          pltpu.VMEM((D, 128), jnp.float32),
          pltpu.VMEM((_L, 128), jnp.float32),
          pltpu.SemaphoreType.DMA,
          pltpu.SemaphoreType.DMA,
      ],
  )
  def emb(tw_hbm, tail_hbm, idx_hbm, out_hbm, pool, staged, tailbuf, rowbuf,
          sem, sem2):
    wid = lax.axis_index("s") * _NC + lax.axis_index("c")
    lanes = lax.iota(jnp.int32, _L)
    hit_id = pool.at[pl.ds(0, nb)]
    hit_pos = pool.at[pl.ds(nb, nb)]
    cid = pool.at[pl.ds(2 * nb, nb)]
    cpos = pool.at[pl.ds(3 * nb, nb)]
    idx_v = pool.at[pl.ds(2 * nb, nb)]

    pltpu.sync_copy(idx_hbm, idx_v)
    c0 = jnp.minimum(wid * sw, c0max)
    lo = wid * sw * 128
    hi = lo + sw * 128

    def scan(j, cnt):
      v = idx_v[pl.ds(j * _L, _L)]
      m = (v >= lo) & (v < hi)
      plsc.store_compressed(hit_id.at[pl.ds(cnt, _L)], v, mask=m)
      plsc.store_compressed(hit_pos.at[pl.ds(cnt, _L)], j * _L + lanes, mask=m)
      return cnt + lax.reduce_max(plsc.all_reduce_population_count(m), (0,))

    cnt = lax.fori_loop(0, B // _L, scan, jnp.int32(0))
    nhit_vecs = (cnt + _L - 1) // _L

    def process(src, width, clo, chi, cnt):
      def cscan(j, cnt2):
        v = hit_id[pl.ds(j * _L, _L)]
        p = hit_pos[pl.ds(j * _L, _L)]
        m = (j * _L + lanes < cnt) & (v >= clo) & (v < chi)
        plsc.store_compressed(cid.at[pl.ds(cnt2, _L)], v, mask=m)
        plsc.store_compressed(cpos.at[pl.ds(cnt2, _L)], p, mask=m)
        return cnt2 + lax.reduce_max(plsc.all_reduce_population_count(m), (0,))

      cnt2 = lax.fori_loop(0, nhit_vecs, cscan, jnp.int32(0))

      def extract(h, carry):
        valid = h * _L + lanes < cnt2
        hv = cid[pl.ds(h * _L, _L)]
        pv = cpos[pl.ds(h * _L, _L)]
        lc = jnp.clip(hv - clo, 0, width - 1)
        pos = jnp.where(valid, pv, B + lanes)
        for d in range(D):
          v = plsc.load_gather(src, [jnp.full((_L,), d, jnp.int32), lc])
          plsc.store_scatter(rowbuf, [lanes, jnp.full((_L,), d, jnp.int32)], v)
        pltpu.async_copy(rowbuf, out_hbm.at[pos], sem2).wait()
        return carry

      lax.fori_loop(0, (cnt2 + _L - 1) // _L, extract, jnp.int32(0))

    def chunk(c, carry):
      cbase = jnp.minimum(c0 + c * _CW, cmax)
      off = pl.multiple_of(cbase * 128, 128)
      copies = [
          pltpu.async_copy(
              tw_hbm.at[pl.ds(8 * r, 8), pl.ds(off, _CW * 128)],
              staged.at[pl.ds(8 * r, 8), :],
              sem,
          )
          for r in range(D // 8)
      ]
      for cp in copies:
        cp.wait()
      clo = cbase * 128
      process(staged, _CW * 128, clo, jnp.minimum(clo + _CW * 128, vmax), cnt)
      return carry

    lax.fori_loop(0, nch, chunk, jnp.int32(0))

    if tail:
      pltpu.sync_copy(tail_hbm, tailbuf)
      process(tailbuf, 128, jnp.int32(vmax), jnp.int32(V), cnt)

  return emb


def kernel(input_ids, weight):
  V, D = weight.shape
  (B,) = input_ids.shape
  emb = _make_kernel(V, D, B)
  vmax = (V // 128) * 128
  tail_t = jnp.pad(weight[vmax:].T, ((0, 0), (0, 128 - (V - vmax))))
  out2 = emb(weight.T, tail_t, input_ids.astype(jnp.int32))
  return out2[:B, :D]
